# Initial kernel scaffold; baseline (speedup 1.0000x reference)
import jax
import jax.numpy as jnp
from jax.experimental import pallas as pl


def kernel(x, edge_index, W1l, W1r, b1, W2l, W2r, b2):
    raise NotImplementedError("write your pallas kernel here")


# speedup vs baseline: 5.2608x; 5.2608x over previous
import functools

import jax
import jax.numpy as jnp
from jax import lax
from jax.experimental import pallas as pl
from jax.experimental.pallas import tpu as pltpu
from jax.experimental.pallas import tpu_sc as plsc

N = 10000
E = 320000
D_IN = 128
HID = 64
D_OUT = 64

NP = 10240
NW = 32
K = 128
STEPS = -(-(E // NW) // K)
EP = NW * STEPS * K
ROWS_PER_TILE = NP // 16
DT = 128


def _make_segsum():
    mesh = plsc.VectorSubcoreMesh(core_axis_name="c", subcore_axis_name="s")

    @functools.partial(
        pl.kernel,
        mesh=mesh,
        out_type=jax.ShapeDtypeStruct((2, NP, DT), jnp.float32),
        scratch_types=[
            pltpu.VMEM((STEPS, K), jnp.int32),
            pltpu.VMEM((STEPS, K), jnp.int32),
            pltpu.VMEM((K, DT), jnp.float32),
            pltpu.VMEM_SHARED((NP, DT), jnp.float32),
            pltpu.SemaphoreType.DMA,
        ],
    )
    def segsum(table_hbm, src_hbm, dst_hbm, zeros_hbm, out_hbm,
               src_v, dst_v, rows_v, acc_sh, sem):
        c = lax.axis_index("c")
        s = lax.axis_index("s")
        wid = s * 2 + c
        r0 = s * ROWS_PER_TILE
        pltpu.sync_copy(zeros_hbm.at[pl.ds(r0, ROWS_PER_TILE)],
                        acc_sh.at[pl.ds(r0, ROWS_PER_TILE)])
        pltpu.sync_copy(src_hbm.at[wid], src_v)
        pltpu.sync_copy(dst_hbm.at[wid], dst_v)
        plsc.subcore_barrier()

        def body(j, carry):
            pltpu.async_copy(table_hbm.at[src_v.at[j]], rows_v, sem).wait()
            pltpu.sync_copy(rows_v, acc_sh.at[dst_v.at[j]], add=True)
            return carry

        lax.fori_loop(0, STEPS, body, 0)
        plsc.subcore_barrier()
        pltpu.sync_copy(acc_sh.at[pl.ds(r0, ROWS_PER_TILE)],
                        out_hbm.at[c, pl.ds(r0, ROWS_PER_TILE)])

    return segsum


_segsum = _make_segsum()


def _mm1_body(x_ref, w_ref, t_ref, xl_ref):
    y = jnp.dot(x_ref[...], w_ref[...], preferred_element_type=jnp.float32)
    xl_ref[...] = y[:, HID:]
    col = lax.broadcasted_iota(jnp.int32, (NP, DT - HID), 1)
    ones = jnp.where(col == 0, 1.0, 0.0).astype(jnp.float32)
    t_ref[...] = jnp.concatenate([y[:, :HID], ones], axis=1)


def _mm2_body(p1_ref, xl_ref, b1_ref, w2_ref, t2_ref, degc_ref):
    acc = p1_ref[0] + p1_ref[1]
    deg = jnp.clip(acc[:, HID:HID + 1], 1.0)
    degc = jnp.broadcast_to(deg, (NP, HID))
    mean1 = acc[:, :HID] / degc
    h = jnp.maximum(xl_ref[...] + mean1 + b1_ref[...], 0.0)
    t2_ref[...] = jnp.dot(h, w2_ref[...], preferred_element_type=jnp.float32)
    degc_ref[...] = degc


def _out_body(p2_ref, t2_ref, degc_ref, b2_ref, o_ref):
    mean2 = (p2_ref[0, :, :D_OUT] + p2_ref[1, :, :D_OUT]) / degc_ref[...]
    lg = t2_ref[:, D_OUT:] + mean2 + b2_ref[...]
    m = jnp.max(lg, axis=1, keepdims=True)
    e = jnp.exp(lg - m)
    o_ref[...] = e / jnp.sum(e, axis=1, keepdims=True)


@jax.jit
def kernel(x, edge_index, W1l, W1r, b1, W2l, W2r, b2):
    xp = jnp.zeros((NP, D_IN), jnp.float32).at[:N].set(x)
    w1c = jnp.concatenate([W1r, W1l], axis=0).T
    w2c = jnp.concatenate([W2r, W2l], axis=0).T
    pad = jnp.full((EP - E,), N, jnp.int32)
    srcp = jnp.concatenate([edge_index[0], pad]).reshape(NW, STEPS, K)
    dstp = jnp.concatenate([edge_index[1], pad]).reshape(NW, STEPS, K)
    zeros = jnp.zeros((NP, DT), jnp.float32)

    t1, xl = pl.pallas_call(
        _mm1_body,
        out_shape=[jax.ShapeDtypeStruct((NP, DT), jnp.float32),
                   jax.ShapeDtypeStruct((NP, HID), jnp.float32)],
    )(xp, w1c)

    p1 = _segsum(t1, srcp, dstp, zeros)

    t2, degc = pl.pallas_call(
        _mm2_body,
        out_shape=[jax.ShapeDtypeStruct((NP, DT), jnp.float32),
                   jax.ShapeDtypeStruct((NP, HID), jnp.float32)],
    )(p1, xl, b1.reshape(1, HID), w2c)

    p2 = _segsum(t2, srcp, dstp, zeros)

    out = pl.pallas_call(
        _out_body,
        out_shape=jax.ShapeDtypeStruct((NP, D_OUT), jnp.float32),
    )(p2, t2, degc, b2.reshape(1, D_OUT))
    return out[:N]
